# baseline (speedup 1.0000x reference)
import functools

import numpy as np
import jax
import jax.numpy as jnp
from jax import lax
from jax.experimental import pallas as pl
from jax.experimental.pallas import tpu as pltpu
from jax.experimental.pallas import tpu_sc as plsc

_BINS = 30
_NC, _NS, _L = 2, 16, 16
_NW = _NC * _NS
_LUT_SIZE = 32768
_BANKS = 4
_SLOTS = 32 * _L

_CHUNK_ROWS = 16
_QPR_SHIFT = 4


@functools.cache
def _neglog_lut() -> jax.Array:
    idx = np.arange(_LUT_SIZE, dtype=np.int64)
    bits = (idx << 15) | (1 << 14)
    vals = bits.astype(np.uint32).view(np.float32).astype(np.float64)
    with np.errstate(divide="ignore"):
        neglog = np.minimum(-np.log(vals), 100.0)
    neglog[0] = 100.0
    neglog[(0x3F800000 >> 15)] = 0.0
    return jnp.asarray(neglog.astype(np.float32))


def _sc_partials(pred: jax.Array, target: jax.Array, sc_rows: int) -> jax.Array:
    rows, cols = pred.shape
    rows_per_w = sc_rows // _NW
    chunks = rows_per_w // _CHUNK_ROWS
    vecs_per_row = cols // _L
    quads = _CHUNK_ROWS * vecs_per_row // _BANKS
    quads_per_row = vecs_per_row // _BANKS
    assert quads_per_row == 1 << _QPR_SHIFT
    mesh = plsc.VectorSubcoreMesh(core_axis_name="c", subcore_axis_name="s")

    @functools.partial(
        pl.kernel,
        out_type=jax.ShapeDtypeStruct((2, _NW, _BANKS * _SLOTS), jnp.float32),
        mesh=mesh,
        compiler_params=pltpu.CompilerParams(needs_layout_passes=False),
        scratch_types=[
            pltpu.VMEM((_LUT_SIZE,), jnp.float32),
            pltpu.VMEM((2, _CHUNK_ROWS, cols), jnp.float32),
            pltpu.VMEM((2, _CHUNK_ROWS, cols), jnp.float32),
            pltpu.VMEM((_BANKS * _SLOTS,), jnp.float32),
            pltpu.VMEM((_BANKS * _SLOTS,), jnp.float32),
            pltpu.SemaphoreType.DMA,
            pltpu.SemaphoreType.DMA,
        ],
    )
    def run(pred_hbm, target_hbm, lut_hbm, out_hbm,
            lut_v, p_v, t_v, nb_v, sb_v, sem0, sem1):
        wid = lax.axis_index("s") * _NC + lax.axis_index("c")
        pltpu.sync_copy(lut_hbm, lut_v)

        zeros = jnp.zeros((_L,), jnp.float32)
        for j in range(_BANKS * _SLOTS // _L):
            nb_v[pl.ds(j * _L, _L)] = zeros
            sb_v[pl.ds(j * _L, _L)] = zeros

        lane = lax.iota(jnp.int32, _L)
        lanes = [lane + jnp.int32(k * _SLOTS) for k in range(_BANKS)]
        ones = jnp.ones((_L,), jnp.float32)
        row0 = wid * rows_per_w
        sems = (sem0, sem1)

        def copies(c, b):
            base = row0 + c * _CHUNK_ROWS
            return (
                pltpu.make_async_copy(
                    pred_hbm.at[pl.ds(base, _CHUNK_ROWS)], p_v.at[b], sems[b]),
                pltpu.make_async_copy(
                    target_hbm.at[pl.ds(base, _CHUNK_ROWS)], t_v.at[b], sems[b]),
            )

        def start(c, b):
            for cp in copies(c, b):
                cp.start()

        def wait(c, b):
            for cp in copies(c, b):
                cp.wait()

        def process(b):
            @plsc.parallel_loop(0, quads, unroll=2)
            def quad_body(v):
                i = lax.shift_right_logical(v, _QPR_SHIFT)
                col0 = (v & (quads_per_row - 1)) * (_BANKS * _L)
                for k in range(_BANKS):
                    off = col0 + k * _L
                    p = p_v[b, i, pl.ds(off, _L)]
                    t = t_v[b, i, pl.ds(off, _L)]
                    g = jnp.abs(p - t)
                    bidx = (g * jnp.float32(_BINS)).astype(jnp.int32)
                    bp = lax.shift_right_logical(
                        lax.bitcast_convert_type(p, jnp.int32), 15)
                    q = jnp.float32(1.0) - p
                    bq = lax.shift_right_logical(
                        lax.bitcast_convert_type(q, jnp.int32), 15)
                    nlp = plsc.load_gather(lut_v, [bp])
                    nlq = plsc.load_gather(lut_v, [bq])
                    bce = nlq + t * (nlp - nlq)
                    slot = bidx * _L + lanes[k]
                    plsc.addupdate_scatter(nb_v, [slot], ones)
                    plsc.addupdate_scatter(sb_v, [slot], bce)

        start(0, 0)

        def pair_body(j, _):
            c0 = 2 * j
            start(c0 + 1, 1)
            wait(c0, 0)
            process(0)

            @pl.when(j < chunks // 2 - 1)
            def _start_next():
                start(c0 + 2, 0)

            wait(c0 + 1, 1)
            process(1)
            return _

        lax.fori_loop(0, chunks // 2, pair_body, 0)

        pltpu.sync_copy(nb_v, out_hbm.at[0, wid])
        pltpu.sync_copy(sb_v, out_hbm.at[1, wid])

    return run(pred, target, _neglog_lut())


_TC_BLOCK_ROWS = 512


def _tc_partials(pred: jax.Array, target: jax.Array, sc_rows: int) -> jax.Array:
    rows, cols = pred.shape
    tc_rows = rows - sc_rows
    nblocks = tc_rows // _TC_BLOCK_ROWS
    block0 = sc_rows // _TC_BLOCK_ROWS

    def body(p_ref, t_ref, out_ref):
        @pl.when(pl.program_id(0) == 0)
        def _init():
            for k in range(2):
                for i in range(32):
                    out_ref[k, i] = jnp.float32(0.0)

        p = p_ref[...]
        t = t_ref[...]
        g = jnp.abs(p - t)
        bidx = (g * jnp.float32(_BINS)).astype(jnp.int32)
        log_p = jnp.clip(jnp.log(p), -100.0, None)
        log_1mp = jnp.clip(jnp.log(1.0 - p), -100.0, None)
        bce = -(t * log_p + (1.0 - t) * log_1mp)
        for i in range(_BINS):
            c = bidx == i
            out_ref[0, i] += jnp.sum(jnp.where(c, 1.0, 0.0))
            out_ref[1, i] += jnp.sum(jnp.where(c, bce, 0.0))

    return pl.pallas_call(
        body,
        grid=(nblocks,),
        in_specs=[
            pl.BlockSpec((_TC_BLOCK_ROWS, cols), lambda j: (block0 + j, 0)),
            pl.BlockSpec((_TC_BLOCK_ROWS, cols), lambda j: (block0 + j, 0)),
        ],
        out_specs=pl.BlockSpec(memory_space=pltpu.SMEM),
        out_shape=jax.ShapeDtypeStruct((2, 32), jnp.float32),
    )(pred, target)


def _combine_body(parts_ref, tc_ref, out_ref):
    nslots = _BANKS * _SLOTS
    acc = jnp.zeros((2, nslots), jnp.float32)
    for w in range(_NW):
        acc = acc + parts_ref[:, w]
    slot_bin = (lax.broadcasted_iota(jnp.int32, (nslots, 32), 0) % _SLOTS) // _L
    bin_col = lax.broadcasted_iota(jnp.int32, (nslots, 32), 1)
    fold = (slot_bin == bin_col).astype(jnp.float32)
    per_bin = jnp.dot(acc, fold) + tc_ref[...]
    counts = per_bin[0:1, :]
    sums = per_bin[1:2, :]
    nonempty = counts > 0.0
    terms = jnp.where(nonempty, sums / jnp.maximum(counts, 1.0), 0.0)
    n = jnp.sum(nonempty.astype(jnp.float32))
    loss = jnp.sum(terms) / jnp.maximum(n, 1.0)
    out_ref[:, :] = jnp.full((1, 1), loss, jnp.float32)


_SC_ROWS = 11264


def kernel(pred, target, batch_size):
    del batch_size
    target = target.astype(jnp.float32)
    sc_parts = _sc_partials(pred, target, _SC_ROWS)
    tc_parts = _tc_partials(pred, target, _SC_ROWS)

    loss = pl.pallas_call(
        _combine_body,
        out_shape=jax.ShapeDtypeStruct((1, 1), jnp.float32),
    )(sc_parts, tc_parts)
    return loss.reshape(())


# speedup vs baseline: 1.3058x; 1.3058x over previous
import functools

import numpy as np
import jax
import jax.numpy as jnp
from jax import lax
from jax.experimental import pallas as pl
from jax.experimental.pallas import tpu as pltpu
from jax.experimental.pallas import tpu_sc as plsc

_BINS = 30
_NC, _NS, _L = 2, 16, 16
_NW = _NC * _NS
_LUT_SIZE = 32768
_BANKS = 4
_SLOTS = 32 * _L

_CHUNK_ROWS = 16
_QPR_SHIFT = 4


@functools.cache
def _neglog_lut() -> jax.Array:
    idx = np.arange(_LUT_SIZE, dtype=np.int64)
    bits = (idx << 15) | (1 << 14)
    vals = bits.astype(np.uint32).view(np.float32).astype(np.float64)
    with np.errstate(divide="ignore"):
        neglog = np.minimum(-np.log(vals), 100.0)
    neglog[0] = 100.0
    neglog[(0x3F800000 >> 15)] = 0.0
    return jnp.asarray(neglog.astype(np.float32))


def _sc_partials(pred: jax.Array, target: jax.Array, sc_rows: int) -> jax.Array:
    rows, cols = pred.shape
    rows_per_w = sc_rows // _NW
    chunks = rows_per_w // _CHUNK_ROWS
    vecs_per_row = cols // _L
    quads = _CHUNK_ROWS * vecs_per_row // _BANKS
    quads_per_row = vecs_per_row // _BANKS
    assert quads_per_row == 1 << _QPR_SHIFT
    mesh = plsc.VectorSubcoreMesh(core_axis_name="c", subcore_axis_name="s")

    @functools.partial(
        pl.kernel,
        out_type=jax.ShapeDtypeStruct((2, _NW, _BANKS * _SLOTS), jnp.float32),
        mesh=mesh,
        compiler_params=pltpu.CompilerParams(needs_layout_passes=False),
        scratch_types=[
            pltpu.VMEM((_LUT_SIZE,), jnp.float32),
            pltpu.VMEM((2, _CHUNK_ROWS, cols), jnp.float32),
            pltpu.VMEM((2, _CHUNK_ROWS, cols), jnp.float32),
            pltpu.VMEM((_BANKS * _SLOTS,), jnp.float32),
            pltpu.VMEM((_BANKS * _SLOTS,), jnp.float32),
            pltpu.SemaphoreType.DMA,
            pltpu.SemaphoreType.DMA,
        ],
    )
    def run(pred_hbm, target_hbm, lut_hbm, out_hbm,
            lut_v, p_v, t_v, nb_v, sb_v, sem0, sem1):
        wid = lax.axis_index("s") * _NC + lax.axis_index("c")
        pltpu.sync_copy(lut_hbm, lut_v)

        zeros = jnp.zeros((_L,), jnp.float32)
        for j in range(_BANKS * _SLOTS // _L):
            nb_v[pl.ds(j * _L, _L)] = zeros
            sb_v[pl.ds(j * _L, _L)] = zeros

        lane = lax.iota(jnp.int32, _L)
        lanes = [lane + jnp.int32(k * _SLOTS) for k in range(_BANKS)]
        ones = jnp.ones((_L,), jnp.float32)
        row0 = wid * rows_per_w
        sems = (sem0, sem1)

        def copies(c, b):
            base = row0 + c * _CHUNK_ROWS
            return (
                pltpu.make_async_copy(
                    pred_hbm.at[pl.ds(base, _CHUNK_ROWS)], p_v.at[b], sems[b]),
                pltpu.make_async_copy(
                    target_hbm.at[pl.ds(base, _CHUNK_ROWS)], t_v.at[b], sems[b]),
            )

        def start(c, b):
            for cp in copies(c, b):
                cp.start()

        def wait(c, b):
            for cp in copies(c, b):
                cp.wait()

        def process(b):
            @plsc.parallel_loop(0, quads, unroll=2)
            def quad_body(v):
                i = lax.shift_right_logical(v, _QPR_SHIFT)
                col0 = (v & (quads_per_row - 1)) * (_BANKS * _L)
                for k in range(_BANKS):
                    off = col0 + k * _L
                    p = p_v[b, i, pl.ds(off, _L)]
                    t = t_v[b, i, pl.ds(off, _L)]
                    g = jnp.abs(p - t)
                    bidx = (g * jnp.float32(_BINS)).astype(jnp.int32)
                    bp = lax.shift_right_logical(
                        lax.bitcast_convert_type(p, jnp.int32), 15)
                    q = jnp.float32(1.0) - p
                    bq = lax.shift_right_logical(
                        lax.bitcast_convert_type(q, jnp.int32), 15)
                    nlp = plsc.load_gather(lut_v, [bp])
                    nlq = plsc.load_gather(lut_v, [bq])
                    bce = nlq + t * (nlp - nlq)
                    slot = bidx * _L + lanes[k]
                    plsc.addupdate_scatter(nb_v, [slot], ones)
                    plsc.addupdate_scatter(sb_v, [slot], bce)

        start(0, 0)

        def pair_body(j, _):
            c0 = 2 * j
            start(c0 + 1, 1)
            wait(c0, 0)
            process(0)

            @pl.when(j < chunks // 2 - 1)
            def _start_next():
                start(c0 + 2, 0)

            wait(c0 + 1, 1)
            process(1)
            return _

        lax.fori_loop(0, chunks // 2, pair_body, 0)

        pltpu.sync_copy(nb_v, out_hbm.at[0, wid])
        pltpu.sync_copy(sb_v, out_hbm.at[1, wid])

    return run(pred, target, _neglog_lut())


_TC_BLOCK_ROWS = 512


def _tc_partials(pred: jax.Array, target: jax.Array, sc_rows: int) -> jax.Array:
    rows, cols = pred.shape
    tc_rows = rows - sc_rows
    nblocks = tc_rows // _TC_BLOCK_ROWS
    block0 = sc_rows // _TC_BLOCK_ROWS

    def body(p_ref, t_ref, out_ref):
        @pl.when(pl.program_id(0) == 0)
        def _init():
            for k in range(2):
                for i in range(32):
                    out_ref[k, i] = jnp.float32(0.0)

        p = p_ref[...]
        t = t_ref[...]
        g = jnp.abs(p - t)
        bidx = (g * jnp.float32(_BINS)).astype(jnp.int32)
        log_p = jnp.clip(jnp.log(p), -100.0, None)
        log_1mp = jnp.clip(jnp.log(1.0 - p), -100.0, None)
        bce = -(t * log_p + (1.0 - t) * log_1mp)
        for i in range(_BINS):
            m = (bidx == i).astype(jnp.float32)
            out_ref[0, i] += jnp.sum(m)
            out_ref[1, i] += jnp.sum(m * bce)

    return pl.pallas_call(
        body,
        grid=(nblocks,),
        in_specs=[
            pl.BlockSpec((_TC_BLOCK_ROWS, cols), lambda j: (block0 + j, 0)),
            pl.BlockSpec((_TC_BLOCK_ROWS, cols), lambda j: (block0 + j, 0)),
        ],
        out_specs=pl.BlockSpec(memory_space=pltpu.SMEM),
        out_shape=jax.ShapeDtypeStruct((2, 32), jnp.float32),
    )(pred, target)


def _combine_body(parts_ref, tc_ref, out_ref):
    nslots = _BANKS * _SLOTS
    acc = jnp.zeros((2, nslots), jnp.float32)
    for w in range(_NW):
        acc = acc + parts_ref[:, w]
    slot_bin = (lax.broadcasted_iota(jnp.int32, (nslots, 32), 0) % _SLOTS) // _L
    bin_col = lax.broadcasted_iota(jnp.int32, (nslots, 32), 1)
    fold = (slot_bin == bin_col).astype(jnp.float32)
    per_bin = jnp.dot(acc, fold) + tc_ref[...]
    counts = per_bin[0:1, :]
    sums = per_bin[1:2, :]
    nonempty = counts > 0.0
    terms = jnp.where(nonempty, sums / jnp.maximum(counts, 1.0), 0.0)
    n = jnp.sum(nonempty.astype(jnp.float32))
    loss = jnp.sum(terms) / jnp.maximum(n, 1.0)
    out_ref[:, :] = jnp.full((1, 1), loss, jnp.float32)


_SC_ROWS = 12288


def kernel(pred, target, batch_size):
    del batch_size
    target = target.astype(jnp.float32)
    sc_parts = _sc_partials(pred, target, _SC_ROWS)
    tc_parts = _tc_partials(pred, target, _SC_ROWS)

    loss = pl.pallas_call(
        _combine_body,
        out_shape=jax.ShapeDtypeStruct((1, 1), jnp.float32),
    )(sc_parts, tc_parts)
    return loss.reshape(())
